# baseline (speedup 1.0000x reference)
import functools

import jax
import jax.numpy as jnp
from jax import lax
from jax.experimental import pallas as pl
from jax.experimental.pallas import tpu as pltpu
from jax.experimental.pallas import tpu_sc as plsc

NBUF = 2
LANE = 16
HC = 2


@functools.partial(
    jax.jit, static_argnames=("batch", "hist", "D", "vocab", "num_cores", "num_subcores")
)
def _sc_embedding_gather(idx_flat, table, *, batch, hist, D, vocab, num_cores, num_subcores):
    mesh = plsc.VectorSubcoreMesh(core_axis_name="c", subcore_axis_name="s")
    num_workers = num_cores * num_subcores
    cols_per_w = batch // num_workers
    idx_per_w = cols_per_w * hist
    n_lb = cols_per_w // LANE
    n_chunks = hist // HC
    assert hist % HC == 0 and cols_per_w % LANE == 0 and D % LANE == 0

    @functools.partial(
        pl.kernel,
        mesh=mesh,
        out_type=jax.ShapeDtypeStruct((hist, D, batch), jnp.float32),
        compiler_params=pltpu.CompilerParams(needs_layout_passes=False),
        scratch_types=[
            pltpu.VMEM((idx_per_w,), jnp.int32),
            pltpu.VMEM((vocab * D,), jnp.float32),
            pltpu.VMEM((NBUF, HC, D, LANE * n_lb), jnp.float32),
            pltpu.SemaphoreType.DMA((NBUF,)),
        ],
    )
    def k(idx_hbm, table_hbm, out_hbm, idx_v, table_v, buf_v, wsem):
        wid = lax.axis_index("s") * num_cores + lax.axis_index("c")
        col_base = wid * cols_per_w
        pltpu.sync_copy(table_hbm, table_v)
        pltpu.sync_copy(idx_hbm.at[pl.ds(col_base * hist, idx_per_w)], idx_v)

        iota = lax.iota(jnp.int32, LANE)
        ibase = [(lb * LANE + iota) * hist for lb in range(n_lb)]

        def fill_chunk(j, b):
            for hh in range(HC):
                h = j * HC + hh
                for lb in range(n_lb):
                    idx16 = plsc.load_gather(idx_v, [ibase[lb] + h])
                    addr16 = idx16 * D
                    for d in range(D):
                        v = plsc.load_gather(table_v, [addr16])
                        buf_v[b, hh, d, pl.ds(lb * LANE, LANE)] = v
                        if d + 1 < D:
                            addr16 = addr16 + 1

        def start_write(j, b):
            pltpu.async_copy(
                buf_v.at[b],
                out_hbm.at[pl.ds(j * HC, HC), :, pl.ds(col_base, cols_per_w)],
                wsem.at[b],
            )

        def wait_write(b):
            pltpu.make_async_copy(
                buf_v.at[b],
                out_hbm.at[pl.ds(0, HC), :, pl.ds(col_base, cols_per_w)],
                wsem.at[b],
            ).wait()

        def body(j, carry):
            b = lax.rem(j, NBUF)

            @pl.when(j >= NBUF)
            def _():
                wait_write(b)

            fill_chunk(j, b)
            start_write(j, b)
            return carry

        lax.fori_loop(0, n_chunks, body, 0)

        for b in range(NBUF):
            wait_write(b)

    return k(idx_flat, table)


def kernel(location, table):
    batch, hist = location.shape
    vocab, D = table.shape
    info = plsc.get_sparse_core_info()
    assert batch % (info.num_cores * info.num_subcores * LANE) == 0
    out_t = _sc_embedding_gather(
        location.astype(jnp.int32).reshape(-1),
        table.astype(jnp.float32).reshape(-1),
        batch=batch,
        hist=hist,
        D=D,
        vocab=vocab,
        num_cores=info.num_cores,
        num_subcores=info.num_subcores,
    )
    return jnp.transpose(out_t, (2, 0, 1))


# speedup vs baseline: 1.6980x; 1.6980x over previous
import functools

import jax
import jax.numpy as jnp
from jax import lax
from jax.experimental import pallas as pl
from jax.experimental.pallas import tpu as pltpu
from jax.experimental.pallas import tpu_sc as plsc

NBUF = 2
LANE = 16
HC = 2


@functools.partial(
    jax.jit, static_argnames=("batch", "hist", "D", "vocab", "num_cores", "num_subcores")
)
def _sc_embedding_gather(idx_flat, table, *, batch, hist, D, vocab, num_cores, num_subcores):
    mesh = plsc.VectorSubcoreMesh(core_axis_name="c", subcore_axis_name="s")
    num_workers = num_cores * num_subcores
    cols_per_w = batch // num_workers
    idx_per_w = cols_per_w * hist
    n_lb = cols_per_w // LANE
    n_chunks = hist // HC
    assert hist % HC == 0 and cols_per_w % LANE == 0 and D % LANE == 0

    @functools.partial(
        pl.kernel,
        mesh=mesh,
        out_type=jax.ShapeDtypeStruct((hist, D, batch), jnp.float32),
        compiler_params=pltpu.CompilerParams(needs_layout_passes=False),
        scratch_types=[
            pltpu.VMEM((idx_per_w,), jnp.int32),
            pltpu.VMEM((vocab * D,), jnp.float32),
            pltpu.VMEM((NBUF, HC, D, LANE * n_lb), jnp.float32),
            pltpu.SemaphoreType.DMA((NBUF,)),
        ],
    )
    def k(idx_hbm, table_hbm, out_hbm, idx_v, table_v, buf_v, wsem):
        wid = lax.axis_index("s") * num_cores + lax.axis_index("c")
        col_base = wid * cols_per_w
        pltpu.sync_copy(table_hbm, table_v)
        pltpu.sync_copy(idx_hbm.at[pl.ds(col_base * hist, idx_per_w)], idx_v)

        iota = lax.iota(jnp.int32, LANE)
        ibase = [(lb * LANE + iota) * hist for lb in range(n_lb)]

        def fill_chunk(j, b):
            for hh in range(HC):
                h = j * HC + hh
                for lb in range(n_lb):
                    idx16 = plsc.load_gather(idx_v, [ibase[lb] + h])
                    addr16 = idx16 * D
                    db = 16
                    for d0 in range(0, D, db):
                        vals = []
                        for d in range(d0, min(d0 + db, D)):
                            vals.append((d, plsc.load_gather(table_v, [addr16])))
                            if d + 1 < D:
                                addr16 = addr16 + 1
                        for d, v in vals:
                            buf_v[b, hh, d, pl.ds(lb * LANE, LANE)] = v

        def start_write(j, b):
            pltpu.async_copy(
                buf_v.at[b],
                out_hbm.at[pl.ds(j * HC, HC), :, pl.ds(col_base, cols_per_w)],
                wsem.at[b],
            )

        def wait_write(b):
            pltpu.make_async_copy(
                buf_v.at[b],
                out_hbm.at[pl.ds(0, HC), :, pl.ds(col_base, cols_per_w)],
                wsem.at[b],
            ).wait()

        def body(j, carry):
            b = lax.rem(j, NBUF)

            @pl.when(j >= NBUF)
            def _():
                wait_write(b)

            fill_chunk(j, b)
            start_write(j, b)
            return carry

        lax.fori_loop(0, n_chunks, body, 0)

        for b in range(NBUF):
            wait_write(b)

    return k(idx_flat, table)


def kernel(location, table):
    batch, hist = location.shape
    vocab, D = table.shape
    info = plsc.get_sparse_core_info()
    assert batch % (info.num_cores * info.num_subcores * LANE) == 0
    out_t = _sc_embedding_gather(
        location.astype(jnp.int32).reshape(-1),
        table.astype(jnp.float32).reshape(-1),
        batch=batch,
        hist=hist,
        D=D,
        vocab=vocab,
        num_cores=info.num_cores,
        num_subcores=info.num_subcores,
    )
    return jnp.transpose(out_t, (2, 0, 1))


# speedup vs baseline: 3.5702x; 2.1027x over previous
import functools

import jax
import jax.numpy as jnp
from jax import lax
from jax.experimental import pallas as pl
from jax.experimental.pallas import tpu as pltpu
from jax.experimental.pallas import tpu_sc as plsc

NBUF = 2
LANE = 16
HC = 2
TPITCH = 72


@functools.partial(
    jax.jit, static_argnames=("batch", "hist", "D", "vocab", "num_cores", "num_subcores")
)
def _sc_embedding_gather(idx_flat, table, *, batch, hist, D, vocab, num_cores, num_subcores):
    mesh = plsc.VectorSubcoreMesh(core_axis_name="c", subcore_axis_name="s")
    num_workers = num_cores * num_subcores
    cols_per_w = batch // num_workers
    idx_per_w = cols_per_w * hist
    n_lb = cols_per_w // LANE
    n_chunks = hist // HC
    assert hist % HC == 0 and cols_per_w % LANE == 0 and D % LANE == 0

    @functools.partial(
        pl.kernel,
        mesh=mesh,
        out_type=jax.ShapeDtypeStruct((hist, D, batch), jnp.float32),
        compiler_params=pltpu.CompilerParams(needs_layout_passes=False),
        scratch_types=[
            pltpu.VMEM((idx_per_w,), jnp.int32),
            pltpu.VMEM((vocab * D,), jnp.float32),
            pltpu.VMEM((vocab * TPITCH,), jnp.float32),
            pltpu.VMEM((NBUF, HC, D, LANE * n_lb), jnp.float32),
            pltpu.SemaphoreType.DMA((NBUF,)),
        ],
    )
    def k(idx_hbm, table_hbm, out_hbm, idx_v, table_v, table_p, buf_v, wsem):
        wid = lax.axis_index("s") * num_cores + lax.axis_index("c")
        col_base = wid * cols_per_w
        pltpu.sync_copy(table_hbm, table_v)
        pltpu.sync_copy(idx_hbm.at[pl.ds(col_base * hist, idx_per_w)], idx_v)

        def repack(r, carry):
            vals = [table_v[pl.ds(r * D + c * LANE, LANE)] for c in range(D // LANE)]
            for c, v in enumerate(vals):
                table_p[pl.ds(r * TPITCH + c * LANE, LANE)] = v
            return carry

        lax.fori_loop(0, vocab, repack, 0)

        iota = lax.iota(jnp.int32, LANE)
        ibase = [(lb * LANE + iota) * hist for lb in range(n_lb)]

        def fill_chunk(j, b):
            for hh in range(HC):
                h = j * HC + hh
                for lb in range(n_lb):
                    idx16 = plsc.load_gather(idx_v, [ibase[lb] + h])
                    addr16 = idx16 * TPITCH
                    db = 16
                    for d0 in range(0, D, db):
                        vals = []
                        for d in range(d0, min(d0 + db, D)):
                            vals.append((d, plsc.load_gather(table_p, [addr16])))
                            if d + 1 < D:
                                addr16 = addr16 + 1
                        for d, v in vals:
                            buf_v[b, hh, d, pl.ds(lb * LANE, LANE)] = v

        def start_write(j, b):
            pltpu.async_copy(
                buf_v.at[b],
                out_hbm.at[pl.ds(j * HC, HC), :, pl.ds(col_base, cols_per_w)],
                wsem.at[b],
            )

        def wait_write(b):
            pltpu.make_async_copy(
                buf_v.at[b],
                out_hbm.at[pl.ds(0, HC), :, pl.ds(col_base, cols_per_w)],
                wsem.at[b],
            ).wait()

        def body(j, carry):
            b = lax.rem(j, NBUF)

            @pl.when(j >= NBUF)
            def _():
                wait_write(b)

            fill_chunk(j, b)
            start_write(j, b)
            return carry

        lax.fori_loop(0, n_chunks, body, 0)

        for b in range(NBUF):
            wait_write(b)

    return k(idx_flat, table)


def kernel(location, table):
    batch, hist = location.shape
    vocab, D = table.shape
    info = plsc.get_sparse_core_info()
    assert batch % (info.num_cores * info.num_subcores * LANE) == 0
    out_t = _sc_embedding_gather(
        location.astype(jnp.int32).reshape(-1),
        table.astype(jnp.float32).reshape(-1),
        batch=batch,
        hist=hist,
        D=D,
        vocab=vocab,
        num_cores=info.num_cores,
        num_subcores=info.num_subcores,
    )
    return jnp.transpose(out_t, (2, 0, 1))


# speedup vs baseline: 3.8771x; 1.0860x over previous
import functools

import jax
import jax.numpy as jnp
from jax import lax
from jax.experimental import pallas as pl
from jax.experimental.pallas import tpu as pltpu
from jax.experimental.pallas import tpu_sc as plsc

NBUF = 2
LANE = 16
HC = 2
TPITCH = 65


@functools.partial(
    jax.jit, static_argnames=("batch", "hist", "D", "vocab", "num_cores", "num_subcores")
)
def _sc_embedding_gather(idx_flat, table, *, batch, hist, D, vocab, num_cores, num_subcores):
    mesh = plsc.VectorSubcoreMesh(core_axis_name="c", subcore_axis_name="s")
    num_workers = num_cores * num_subcores
    cols_per_w = batch // num_workers
    idx_per_w = cols_per_w * hist
    n_lb = cols_per_w // LANE
    n_chunks = hist // HC
    assert hist % HC == 0 and cols_per_w % LANE == 0 and D % LANE == 0

    @functools.partial(
        pl.kernel,
        mesh=mesh,
        out_type=jax.ShapeDtypeStruct((hist, D, batch), jnp.float32),
        compiler_params=pltpu.CompilerParams(needs_layout_passes=False),
        scratch_types=[
            pltpu.VMEM((idx_per_w,), jnp.int32),
            pltpu.VMEM((vocab * D,), jnp.float32),
            pltpu.VMEM((vocab * TPITCH,), jnp.float32),
            pltpu.VMEM((NBUF, HC, D, LANE * n_lb), jnp.float32),
            pltpu.SemaphoreType.DMA((NBUF,)),
        ],
    )
    def k(idx_hbm, table_hbm, out_hbm, idx_v, table_v, table_p, buf_v, wsem):
        wid = lax.axis_index("s") * num_cores + lax.axis_index("c")
        col_base = wid * cols_per_w
        pltpu.sync_copy(table_hbm, table_v)
        pltpu.sync_copy(idx_hbm.at[pl.ds(col_base * hist, idx_per_w)], idx_v)

        def repack(r, carry):
            vals = [table_v[pl.ds(r * D + c * LANE, LANE)] for c in range(D // LANE)]
            for c, v in enumerate(vals):
                table_p[pl.ds(r * TPITCH + c * LANE, LANE)] = v
            return carry

        lax.fori_loop(0, vocab, repack, 0)

        iota = lax.iota(jnp.int32, LANE)
        ibase = [(lb * LANE + iota) * hist for lb in range(n_lb)]

        def fill_chunk(j, b):
            for hh in range(HC):
                h = j * HC + hh
                for lb in range(n_lb):
                    idx16 = plsc.load_gather(idx_v, [ibase[lb] + h])
                    addr16 = idx16 * TPITCH
                    db = 16
                    for d0 in range(0, D, db):
                        vals = []
                        for d in range(d0, min(d0 + db, D)):
                            vals.append((d, plsc.load_gather(table_p, [addr16])))
                            if d + 1 < D:
                                addr16 = addr16 + 1
                        for d, v in vals:
                            buf_v[b, hh, d, pl.ds(lb * LANE, LANE)] = v

        def start_write(j, b):
            pltpu.async_copy(
                buf_v.at[b],
                out_hbm.at[pl.ds(j * HC, HC), :, pl.ds(col_base, cols_per_w)],
                wsem.at[b],
            )

        def wait_write(b):
            pltpu.make_async_copy(
                buf_v.at[b],
                out_hbm.at[pl.ds(0, HC), :, pl.ds(col_base, cols_per_w)],
                wsem.at[b],
            ).wait()

        def body(j, carry):
            b = lax.rem(j, NBUF)

            @pl.when(j >= NBUF)
            def _():
                wait_write(b)

            fill_chunk(j, b)
            start_write(j, b)
            return carry

        lax.fori_loop(0, n_chunks, body, 0)

        for b in range(NBUF):
            wait_write(b)

    return k(idx_flat, table)


def kernel(location, table):
    batch, hist = location.shape
    vocab, D = table.shape
    info = plsc.get_sparse_core_info()
    assert batch % (info.num_cores * info.num_subcores * LANE) == 0
    out_t = _sc_embedding_gather(
        location.astype(jnp.int32).reshape(-1),
        table.astype(jnp.float32).reshape(-1),
        batch=batch,
        hist=hist,
        D=D,
        vocab=vocab,
        num_cores=info.num_cores,
        num_subcores=info.num_subcores,
    )
    return jnp.transpose(out_t, (2, 0, 1))
